# baseline (speedup 1.0000x reference)
import functools

import jax
import jax.numpy as jnp
from jax import lax
from jax.experimental import pallas as pl
from jax.experimental.pallas import tpu as pltpu
from jax.experimental.pallas import tpu_sc as plsc

N = 50000
E = 800000
H = 64
N_CLASS = 10
R = 5000

NC = 2
NS = 16
L = 16

HALF = N // 2
ACC_ROWS = 25088
STRIPE = ACC_ROWS // NS
GARBAGE = HALF
CHUNK = 128
SPAN_Q = NS * CHUNK
CAP = 800768
CAPT = CAP // CHUNK

DEG_PAD = 51200
DEG_STRIPE = DEG_PAD // NS
DEG_GARBAGE = 50100
E_PAD = 819200
DTRIPS = 25

_MESH = plsc.VectorSubcoreMesh(core_axis_name="c", subcore_axis_name="s",
                               num_cores=NC, num_subcores=NS)


def _zero_vmem_2d(ref, rows):
    z16 = jnp.zeros((L,), jnp.float32)

    def body(r, _):
        for jj in range(4):
            ref[r, pl.ds(L * jj, L)] = z16
        return 0

    lax.fori_loop(0, rows, body, 0)


def _zero_vmem_1d(ref, words):
    z16 = jnp.zeros((L,), jnp.float32)

    def body(t, _):
        ref[pl.ds(L * t, L)] = z16
        return 0

    lax.fori_loop(0, words // L, body, 0)


def _deg_body(srcp, dstp, degp, acc0, acc1, idx, ones_v, zbuf, sem):
    c = lax.axis_index("c")
    s = lax.axis_index("s")
    wid = c * NS + s
    accs = (acc0, acc1)

    _zero_vmem_1d(zbuf, DEG_STRIPE)
    for a in range(2):
        pltpu.sync_copy(zbuf, accs[a].at[pl.ds(s * DEG_STRIPE, DEG_STRIPE)])
    one16 = jnp.ones((L,), jnp.float32)
    for t in range(128 // L):
        ones_v[pl.ds(L * t, L)] = one16
    plsc.subcore_barrier()

    def trip(t, _):
        row = wid * (DTRIPS * 8) + 8 * t
        pltpu.sync_copy(srcp.at[pl.ds(row, 8)], idx.at[0])
        pltpu.sync_copy(dstp.at[pl.ds(row, 8)], idx.at[1])
        for a in range(2):
            for j in range(8):
                pltpu.async_copy(ones_v, accs[a].at[idx.at[a, j]], sem,
                                 add=True)
        for a in range(2):
            for j in range(8):
                pltpu.make_async_copy(ones_v, accs[a].at[idx.at[a, j]],
                                      sem).wait()
        return 0

    lax.fori_loop(0, DTRIPS, trip, 0)
    plsc.subcore_barrier()
    for a in range(2):
        off = (c * 2 + a) * DEG_PAD + s * DEG_STRIPE
        pltpu.sync_copy(accs[a].at[pl.ds(s * DEG_STRIPE, DEG_STRIPE)],
                        degp.at[pl.ds(off, DEG_STRIPE)])


def _degrees(src, dst):
    fill = jnp.full((E_PAD - E,), DEG_GARBAGE, jnp.int32)
    srcp = jnp.concatenate([src, fill]).reshape(E_PAD // 128, 128)
    dstp = jnp.concatenate([dst, fill]).reshape(E_PAD // 128, 128)
    degp = pl.kernel(
        _deg_body,
        out_type=jax.ShapeDtypeStruct((4 * DEG_PAD,), jnp.float32),
        mesh=_MESH,
        scratch_types=[
            pltpu.VMEM_SHARED((DEG_PAD,), jnp.float32),
            pltpu.VMEM_SHARED((DEG_PAD,), jnp.float32),
            pltpu.VMEM((2, 8, 128), jnp.int32),
            pltpu.VMEM((128,), jnp.float32),
            pltpu.VMEM((DEG_STRIPE,), jnp.float32),
            pltpu.SemaphoreType.DMA,
        ],
    )(srcp, dstp)
    degp = degp.reshape(4, DEG_PAD)
    deg_out = degp[0, :N] + degp[2, :N]
    deg_in = degp[1, :N] + degp[3, :N]
    return deg_out, deg_in


def _spmm_body(u_hbm, idxc, nc_hbm, agg, acc, ibuf, rows, zrow, ncv,
               gsem0, gsem1, gsem2, ssem0, ssem1, ssem2,
               isem0, isem1, isem2, isem3, isem4, isem5):
    c = lax.axis_index("c")
    s = lax.axis_index("s")
    gsem = (gsem0, gsem1, gsem2)
    ssem = (ssem0, ssem1, ssem2)
    isem = (isem0, isem1, isem2, isem3, isem4, isem5)

    _zero_vmem_2d(zrow, 32)
    base = s * STRIPE
    for q in range(STRIPE // 32):
        pltpu.sync_copy(zrow, acc.at[pl.ds(base + 32 * q, 32), :])
    plsc.subcore_barrier()

    pltpu.sync_copy(nc_hbm.at[pl.ds(c * L, L)], ncv)
    n_c = ncv[pl.ds(0, L)][0]
    T = lax.div(n_c + (SPAN_Q - 1), SPAN_Q)
    q0 = s * T

    def fire_idx(k, tt):
        pltpu.async_copy(idxc.at[c].at[q0 + tt], ibuf.at[k], isem[k])

    def drain_idx(k):
        pltpu.make_async_copy(idxc.at[c].at[0], ibuf.at[k], isem[k]).wait()

    def fire_gather(b, k):
        pltpu.async_copy(u_hbm.at[ibuf.at[k, 0]], rows.at[b], gsem[b])

    def drain_gather(b):
        pltpu.make_async_copy(u_hbm.at[pl.ds(0, CHUNK)], rows.at[b],
                              gsem[b]).wait()

    def fire_scatter(b, k):
        pltpu.async_copy(rows.at[b], acc.at[ibuf.at[k, 1]], ssem[b],
                         add=True)

    def drain_scatter(b):
        pltpu.make_async_copy(u_hbm.at[pl.ds(0, CHUNK)], rows.at[b],
                              ssem[b]).wait()

    for t0 in range(3):
        @pl.when(T >= t0 + 1)
        def _(t0=t0):
            fire_idx(t0, t0)

    def six(g, _):
        for k in range(6):
            tt = 6 * g + k
            b = k % 3

            @pl.when(tt < T)
            def _(k=k, b=b, tt=tt):
                @pl.when(tt >= 3)
                def _():
                    drain_scatter(b)

                drain_idx(k)
                fire_gather(b, k)

                @pl.when(tt >= 2)
                def _():
                    drain_gather((b - 2) % 3)
                    fire_scatter((b - 2) % 3, (k - 2) % 6)

                @pl.when(tt + 3 < T)
                def _():
                    fire_idx((k + 3) % 6, tt + 3)

        return 0

    lax.fori_loop(0, lax.div(T + 5, 6), six, 0)

    for k in range(6):
        @pl.when(jnp.logical_and(T >= 1, lax.rem(T - 1, 6) == k))
        def _(k=k):
            b1 = k % 3
            b2 = (k - 1) % 3
            b3 = (k - 2) % 3

            @pl.when(T >= 3)
            def _():
                drain_scatter(b3)

            @pl.when(T >= 2)
            def _():
                drain_gather(b2)
                fire_scatter(b2, (k - 1) % 6)

            drain_gather(b1)
            fire_scatter(b1, k)

            @pl.when(T >= 2)
            def _():
                drain_scatter(b2)

            drain_scatter(b1)

    plsc.subcore_barrier()
    sl = pl.ds(s * STRIPE, STRIPE)
    pltpu.sync_copy(acc.at[sl, :], agg.at[c].at[sl, :])


def _make_spmm():
    return pl.kernel(
        _spmm_body,
        out_type=jax.ShapeDtypeStruct((NC, ACC_ROWS, H), jnp.float32),
        mesh=_MESH,
        scratch_types=[
            pltpu.VMEM_SHARED((ACC_ROWS, H), jnp.float32),
            pltpu.VMEM((6, 2, 128), jnp.int32),
            pltpu.VMEM((3, CHUNK, H), jnp.float32),
            pltpu.VMEM((32, H), jnp.float32),
            pltpu.VMEM((L,), jnp.int32),
            pltpu.SemaphoreType.DMA,
            pltpu.SemaphoreType.DMA,
            pltpu.SemaphoreType.DMA,
            pltpu.SemaphoreType.DMA,
            pltpu.SemaphoreType.DMA,
            pltpu.SemaphoreType.DMA,
            pltpu.SemaphoreType.DMA,
            pltpu.SemaphoreType.DMA,
            pltpu.SemaphoreType.DMA,
            pltpu.SemaphoreType.DMA,
            pltpu.SemaphoreType.DMA,
            pltpu.SemaphoreType.DMA,
        ],
        compiler_params=pltpu.CompilerParams(use_tc_tiling_on_sc=False),
    )


def _edge_prep(src, dst):
    key = (dst >= HALF).astype(jnp.int32)
    n0 = E - jnp.sum(key)
    n1 = E - n0
    dstl = dst - HALF * key
    _, src_s, dstl_s = lax.sort((key, src, dstl), num_keys=1,
                                is_stable=True)

    iota = jnp.arange(CAP, dtype=jnp.int32)
    src_big = jnp.concatenate([src_s, jnp.zeros((CAP,), jnp.int32)])
    dst_big = jnp.concatenate([dstl_s,
                               jnp.full((CAP,), GARBAGE, jnp.int32)])
    r0_src = jnp.where(iota < n0, src_big[:CAP], 0)
    r0_dst = jnp.where(iota < n0, dst_big[:CAP], GARBAGE)
    r1_src = jnp.where(iota < n1,
                       lax.dynamic_slice(src_big, (n0,), (CAP,)), 0)
    r1_dst = jnp.where(iota < n1,
                       lax.dynamic_slice(dst_big, (n0,), (CAP,)), GARBAGE)

    def comb(rs, rd):
        return jnp.stack([rs.reshape(CAPT, 128),
                          rd.reshape(CAPT, 128)], axis=1)

    idxc = jnp.stack([comb(r0_src, r0_dst), comb(r1_src, r1_dst)])
    nc = jnp.concatenate([jnp.full((L,), n0, jnp.int32),
                          jnp.full((L,), n1, jnp.int32)])
    return idxc, nc


_AGG_SPEC = pl.BlockSpec((1, R, H), lambda i: (i // 5, i % 5, 0))


def _tc_in_body(x_ref, cout_ref, w_ref, u_ref):
    x = x_ref[...]
    u_ref[...] = jnp.dot(x * cout_ref[...], w_ref[...],
                         preferred_element_type=jnp.float32)


def _tc_in(x16, cout, w16):
    return pl.pallas_call(
        _tc_in_body,
        grid=(N // R,),
        in_specs=[
            pl.BlockSpec((R, 16), lambda i: (i, 0)),
            pl.BlockSpec((R, 1), lambda i: (i, 0)),
            pl.BlockSpec((16, H), lambda i: (0, 0)),
        ],
        out_specs=pl.BlockSpec((R, H), lambda i: (i, 0)),
        out_shape=jax.ShapeDtypeStruct((N, H), jnp.float32),
    )(x16, cout, w16)


def _tc_step_body(agg_ref, cin_ref, cout_ref, b_ref, w_ref, res_ref,
                  h_ref, u_ref, *, residual):
    h = jnp.maximum(cin_ref[...] * agg_ref[0] + b_ref[...], 0.0)
    if residual:
        h = h + res_ref[...]
    h_ref[...] = h
    u_ref[...] = jnp.dot(h * cout_ref[...], w_ref[...],
                         preferred_element_type=jnp.float32)


def _tc_step(agg, cin, cout, b, w_next, res):
    residual = res is not None
    if res is None:
        res = cin
    res_spec = (pl.BlockSpec((R, H), lambda i: (i, 0)) if residual
                else pl.BlockSpec((R, 1), lambda i: (i, 0)))
    return pl.pallas_call(
        functools.partial(_tc_step_body, residual=residual),
        grid=(N // R,),
        in_specs=[
            _AGG_SPEC,
            pl.BlockSpec((R, 1), lambda i: (i, 0)),
            pl.BlockSpec((R, 1), lambda i: (i, 0)),
            pl.BlockSpec((1, H), lambda i: (0, 0)),
            pl.BlockSpec((H, H), lambda i: (0, 0)),
            res_spec,
        ],
        out_specs=[
            pl.BlockSpec((R, H), lambda i: (i, 0)),
            pl.BlockSpec((R, H), lambda i: (i, 0)),
        ],
        out_shape=[
            jax.ShapeDtypeStruct((N, H), jnp.float32),
            jax.ShapeDtypeStruct((N, H), jnp.float32),
        ],
    )(agg, cin, cout, b, w_next, res)


def _tc_last_body(agg_ref, cin_ref, b_ref, res_ref, h_ref, *, residual):
    h = jnp.maximum(cin_ref[...] * agg_ref[0] + b_ref[...], 0.0)
    if residual:
        h = h + res_ref[...]
    h_ref[...] = h


def _tc_last(agg, cin, b, res):
    residual = res is not None
    if res is None:
        res = cin
    res_spec = (pl.BlockSpec((R, H), lambda i: (i, 0)) if residual
                else pl.BlockSpec((R, 1), lambda i: (i, 0)))
    return pl.pallas_call(
        functools.partial(_tc_last_body, residual=residual),
        grid=(N // R,),
        in_specs=[
            _AGG_SPEC,
            pl.BlockSpec((R, 1), lambda i: (i, 0)),
            pl.BlockSpec((1, H), lambda i: (0, 0)),
            res_spec,
        ],
        out_specs=pl.BlockSpec((R, H), lambda i: (i, 0)),
        out_shape=jax.ShapeDtypeStruct((N, H), jnp.float32),
    )(agg, cin, b, res)


def _readout_body(h1_ref, h2_ref, wfc_ref, bfc_ref, out_ref, acc_ref):
    i = pl.program_id(0)

    @pl.when(i == 0)
    def _init():
        acc_ref[...] = jnp.zeros_like(acc_ref)

    acc_ref[...] += jnp.sum(h1_ref[...] + h2_ref[...], axis=0,
                            keepdims=True)

    @pl.when(i == pl.num_programs(0) - 1)
    def _fin():
        readout = acc_ref[...] / (2.0 * N)
        logits = jnp.dot(readout, wfc_ref[...],
                         preferred_element_type=jnp.float32) + bfc_ref[...]
        m = jnp.max(logits, axis=1, keepdims=True)
        z = logits - m
        out_ref[...] = z - jnp.log(jnp.sum(jnp.exp(z), axis=1,
                                           keepdims=True))


def _readout(h1, h2, wfc, bfc):
    out = pl.pallas_call(
        _readout_body,
        grid=(N // R,),
        in_specs=[
            pl.BlockSpec((R, H), lambda i: (i, 0)),
            pl.BlockSpec((R, H), lambda i: (i, 0)),
            pl.BlockSpec((H, N_CLASS), lambda i: (0, 0)),
            pl.BlockSpec((1, N_CLASS), lambda i: (0, 0)),
        ],
        out_specs=pl.BlockSpec((1, N_CLASS), lambda i: (0, 0)),
        out_shape=jax.ShapeDtypeStruct((1, N_CLASS), jnp.float32),
        scratch_shapes=[pltpu.VMEM((1, H), jnp.float32)],
    )(h1, h2, wfc, bfc)
    return out[0]


def kernel(x_angle, x_distance, edge_index, W_in_a, b_in_a, W_hid_a,
           b_hid_a, W_in_d, b_in_d, W_hid_d, b_hid_d, W_fc, b_fc):
    src = edge_index[0]
    dst = edge_index[1]
    deg_out, deg_in = _degrees(src, dst)
    c_out = lax.rsqrt(jnp.clip(deg_out, 1.0))[:, None]
    c_in = lax.rsqrt(jnp.clip(deg_in, 1.0))[:, None]

    idxc, nc = _edge_prep(src, dst)
    spmm_call = _make_spmm()

    def _spmm(u):
        return spmm_call(u, idxc, nc)

    def pad16(x):
        return jnp.pad(x, ((0, 0), (0, 16 - x.shape[1])))

    xa16 = pad16(x_angle)
    xd16 = pad16(x_distance)
    Wina16 = jnp.pad(W_in_a, ((0, 16 - W_in_a.shape[0]), (0, 0)))
    Wind16 = jnp.pad(W_in_d, ((0, 16 - W_in_d.shape[0]), (0, 0)))

    u_a = _tc_in(xa16, c_out, Wina16)
    u_d = _tc_in(xd16, c_out, Wind16)
    h_a = None
    h_d = None
    for p in range(9):
        agg = _spmm(u_a)
        res = h_a if (p >= 1 and (p - 1) not in (3, 7)) else None
        b = b_in_a[None, :] if p == 0 else b_hid_a[p - 1][None, :]
        h_a, u_a = _tc_step(agg, c_in, c_out, b, W_hid_a[p], res)

        agg = _spmm(u_d)
        res = h_d if (p >= 1 and (p - 1) not in (3, 7)) else None
        b = b_in_d[None, :] if p == 0 else b_hid_d[p - 1][None, :]
        if p < 8:
            h_d, u_d = _tc_step(agg, c_in, c_out, b, W_hid_d[p], res)
        else:
            h_d = _tc_last(agg, c_in, b, res)

    for p in range(9, 12):
        agg = _spmm(u_a)
        res = h_a
        b = b_hid_a[p - 1][None, :]
        w_next = W_hid_a[p] if p < 11 else W_hid_d[8]
        h_a, u_a = _tc_step(agg, c_in, c_out, b, w_next, res)

    h1 = h_a
    u_d = u_a
    for p in range(9, 12):
        agg = _spmm(u_d)
        res = h_d
        b = b_hid_d[p - 1][None, :]
        if p < 11:
            h_d, u_d = _tc_step(agg, c_in, c_out, b, W_hid_d[p], res)
        else:
            h_d = _tc_last(agg, c_in, b, res)

    return _readout(h1, h_d, W_fc, b_fc[None, :])


# speedup vs baseline: 1.0206x; 1.0206x over previous
import functools

import jax
import jax.numpy as jnp
from jax import lax
from jax.experimental import pallas as pl
from jax.experimental.pallas import tpu as pltpu
from jax.experimental.pallas import tpu_sc as plsc

N = 50000
E = 800000
H = 64
N_CLASS = 10
R = 5000

NC = 2
NS = 16
L = 16

HALF = N // 2
ACC_ROWS = 25088
STRIPE = ACC_ROWS // NS
GARBAGE = HALF
CHUNK = 128
SPAN_Q = NS * CHUNK
CAP = 800768
CAPT = CAP // CHUNK

DEG_PAD = 51200
DEG_STRIPE = DEG_PAD // NS
DEG_GARBAGE = 50100
E_PAD = 819200
DTRIPS = 25

_MESH = plsc.VectorSubcoreMesh(core_axis_name="c", subcore_axis_name="s",
                               num_cores=NC, num_subcores=NS)


def _zero_vmem_2d(ref, rows):
    z16 = jnp.zeros((L,), jnp.float32)

    def body(r, _):
        for jj in range(4):
            ref[r, pl.ds(L * jj, L)] = z16
        return 0

    lax.fori_loop(0, rows, body, 0)


def _zero_vmem_1d(ref, words):
    z16 = jnp.zeros((L,), jnp.float32)

    def body(t, _):
        ref[pl.ds(L * t, L)] = z16
        return 0

    lax.fori_loop(0, words // L, body, 0)


def _deg_body(srcp, dstp, degp, acc0, acc1, idx, ones_v, zbuf, sem):
    c = lax.axis_index("c")
    s = lax.axis_index("s")
    wid = c * NS + s
    accs = (acc0, acc1)

    _zero_vmem_1d(zbuf, DEG_STRIPE)
    for a in range(2):
        pltpu.sync_copy(zbuf, accs[a].at[pl.ds(s * DEG_STRIPE, DEG_STRIPE)])
    one16 = jnp.ones((L,), jnp.float32)
    for t in range(128 // L):
        ones_v[pl.ds(L * t, L)] = one16
    plsc.subcore_barrier()

    def trip(t, _):
        row = wid * (DTRIPS * 8) + 8 * t
        pltpu.sync_copy(srcp.at[pl.ds(row, 8)], idx.at[0])
        pltpu.sync_copy(dstp.at[pl.ds(row, 8)], idx.at[1])
        for a in range(2):
            for j in range(8):
                pltpu.async_copy(ones_v, accs[a].at[idx.at[a, j]], sem,
                                 add=True)
        for a in range(2):
            for j in range(8):
                pltpu.make_async_copy(ones_v, accs[a].at[idx.at[a, j]],
                                      sem).wait()
        return 0

    lax.fori_loop(0, DTRIPS, trip, 0)
    plsc.subcore_barrier()
    for a in range(2):
        off = (c * 2 + a) * DEG_PAD + s * DEG_STRIPE
        pltpu.sync_copy(accs[a].at[pl.ds(s * DEG_STRIPE, DEG_STRIPE)],
                        degp.at[pl.ds(off, DEG_STRIPE)])


def _degrees(src, dst):
    fill = jnp.full((E_PAD - E,), DEG_GARBAGE, jnp.int32)
    srcp = jnp.concatenate([src, fill]).reshape(E_PAD // 128, 128)
    dstp = jnp.concatenate([dst, fill]).reshape(E_PAD // 128, 128)
    degp = pl.kernel(
        _deg_body,
        out_type=jax.ShapeDtypeStruct((4 * DEG_PAD,), jnp.float32),
        mesh=_MESH,
        scratch_types=[
            pltpu.VMEM_SHARED((DEG_PAD,), jnp.float32),
            pltpu.VMEM_SHARED((DEG_PAD,), jnp.float32),
            pltpu.VMEM((2, 8, 128), jnp.int32),
            pltpu.VMEM((128,), jnp.float32),
            pltpu.VMEM((DEG_STRIPE,), jnp.float32),
            pltpu.SemaphoreType.DMA,
        ],
    )(srcp, dstp)
    degp = degp.reshape(4, DEG_PAD)
    deg_out = degp[0, :N] + degp[2, :N]
    deg_in = degp[1, :N] + degp[3, :N]
    return deg_out, deg_in


def _spmm_body(u_hbm, idxc, nc_hbm, agg, acc, ibuf, rows, zrow, ncv,
               gsem0, gsem1, gsem2, ssem0, ssem1, ssem2,
               isem0, isem1, isem2, isem3, isem4, isem5):
    c = lax.axis_index("c")
    s = lax.axis_index("s")
    gsem = (gsem0, gsem1, gsem2)
    ssem = (ssem0, ssem1, ssem2)
    isem = (isem0, isem1, isem2, isem3, isem4, isem5)

    _zero_vmem_2d(zrow, 32)
    base = s * STRIPE
    for q in range(STRIPE // 32):
        pltpu.sync_copy(zrow, acc.at[pl.ds(base + 32 * q, 32), :])
    plsc.subcore_barrier()

    pltpu.sync_copy(nc_hbm.at[pl.ds(c * L, L)], ncv)
    n_c = ncv[pl.ds(0, L)][0]
    T = lax.div(n_c + (SPAN_Q - 1), SPAN_Q)
    q0 = s * T

    def fire_idx(k, tt):
        pltpu.async_copy(idxc.at[c].at[q0 + tt], ibuf.at[k], isem[k])

    def drain_idx(k):
        pltpu.make_async_copy(idxc.at[c].at[0], ibuf.at[k], isem[k]).wait()

    def fire_gather(b, k):
        pltpu.async_copy(u_hbm.at[ibuf.at[k, 0]], rows.at[b], gsem[b])

    def drain_gather(b):
        pltpu.make_async_copy(u_hbm.at[pl.ds(0, CHUNK)], rows.at[b],
                              gsem[b]).wait()

    def fire_scatter(b, k):
        pltpu.async_copy(rows.at[b], acc.at[ibuf.at[k, 1]], ssem[b],
                         add=True)

    def drain_scatter(b):
        pltpu.make_async_copy(u_hbm.at[pl.ds(0, CHUNK)], rows.at[b],
                              ssem[b]).wait()

    for t0 in range(3):
        @pl.when(T >= t0 + 1)
        def _(t0=t0):
            fire_idx(t0, t0)

    def six(g, _):
        for k in range(6):
            tt = 6 * g + k
            b = k % 3

            @pl.when(tt < T)
            def _(k=k, b=b, tt=tt):
                @pl.when(tt >= 3)
                def _():
                    drain_scatter(b)

                drain_idx(k)
                fire_gather(b, k)

                @pl.when(tt >= 2)
                def _():
                    drain_gather((b - 2) % 3)
                    fire_scatter((b - 2) % 3, (k - 2) % 6)

                @pl.when(tt + 3 < T)
                def _():
                    fire_idx((k + 3) % 6, tt + 3)

        return 0

    lax.fori_loop(0, lax.div(T + 5, 6), six, 0)

    for k in range(6):
        @pl.when(jnp.logical_and(T >= 1, lax.rem(T - 1, 6) == k))
        def _(k=k):
            b1 = k % 3
            b2 = (k - 1) % 3
            b3 = (k - 2) % 3

            @pl.when(T >= 3)
            def _():
                drain_scatter(b3)

            @pl.when(T >= 2)
            def _():
                drain_gather(b2)
                fire_scatter(b2, (k - 1) % 6)

            drain_gather(b1)
            fire_scatter(b1, k)

            @pl.when(T >= 2)
            def _():
                drain_scatter(b2)

            drain_scatter(b1)

    plsc.subcore_barrier()
    sl = pl.ds(s * STRIPE, STRIPE)
    pltpu.sync_copy(acc.at[sl, :], agg.at[c].at[sl, :])


def _make_spmm():
    return pl.kernel(
        _spmm_body,
        out_type=jax.ShapeDtypeStruct((NC, ACC_ROWS, H), jnp.float32),
        mesh=_MESH,
        scratch_types=[
            pltpu.VMEM_SHARED((ACC_ROWS, H), jnp.float32),
            pltpu.VMEM((6, 2, 128), jnp.int32),
            pltpu.VMEM((3, CHUNK, H), jnp.float32),
            pltpu.VMEM((32, H), jnp.float32),
            pltpu.VMEM((L,), jnp.int32),
            pltpu.SemaphoreType.DMA,
            pltpu.SemaphoreType.DMA,
            pltpu.SemaphoreType.DMA,
            pltpu.SemaphoreType.DMA,
            pltpu.SemaphoreType.DMA,
            pltpu.SemaphoreType.DMA,
            pltpu.SemaphoreType.DMA,
            pltpu.SemaphoreType.DMA,
            pltpu.SemaphoreType.DMA,
            pltpu.SemaphoreType.DMA,
            pltpu.SemaphoreType.DMA,
            pltpu.SemaphoreType.DMA,
        ],
        compiler_params=pltpu.CompilerParams(use_tc_tiling_on_sc=False),
    )


def _edge_prep(src, dst):
    key = (dst >= HALF).astype(jnp.int32)
    n0 = E - jnp.sum(key)
    n1 = E - n0
    order = jnp.argsort(key)
    src_s = src[order]
    dstl_s = (dst - HALF * key)[order]

    iota = jnp.arange(CAP, dtype=jnp.int32)
    src_big = jnp.concatenate([src_s, jnp.zeros((CAP,), jnp.int32)])
    dst_big = jnp.concatenate([dstl_s,
                               jnp.full((CAP,), GARBAGE, jnp.int32)])
    r0_src = jnp.where(iota < n0, src_big[:CAP], 0)
    r0_dst = jnp.where(iota < n0, dst_big[:CAP], GARBAGE)
    r1_src = jnp.where(iota < n1,
                       lax.dynamic_slice(src_big, (n0,), (CAP,)), 0)
    r1_dst = jnp.where(iota < n1,
                       lax.dynamic_slice(dst_big, (n0,), (CAP,)), GARBAGE)

    def comb(rs, rd):
        return jnp.stack([rs.reshape(CAPT, 128),
                          rd.reshape(CAPT, 128)], axis=1)

    idxc = jnp.stack([comb(r0_src, r0_dst), comb(r1_src, r1_dst)])
    nc = jnp.concatenate([jnp.full((L,), n0, jnp.int32),
                          jnp.full((L,), n1, jnp.int32)])
    return idxc, nc


_AGG_SPEC = pl.BlockSpec((1, R, H), lambda i: (i // 5, i % 5, 0))


def _tc_in_body(x_ref, cout_ref, w_ref, u_ref):
    x = x_ref[...]
    u_ref[...] = jnp.dot(x * cout_ref[...], w_ref[...],
                         preferred_element_type=jnp.float32)


def _tc_in(x16, cout, w16):
    return pl.pallas_call(
        _tc_in_body,
        grid=(N // R,),
        in_specs=[
            pl.BlockSpec((R, 16), lambda i: (i, 0)),
            pl.BlockSpec((R, 1), lambda i: (i, 0)),
            pl.BlockSpec((16, H), lambda i: (0, 0)),
        ],
        out_specs=pl.BlockSpec((R, H), lambda i: (i, 0)),
        out_shape=jax.ShapeDtypeStruct((N, H), jnp.float32),
    )(x16, cout, w16)


def _tc_step_body(agg_ref, cin_ref, cout_ref, b_ref, w_ref, res_ref,
                  h_ref, u_ref, *, residual):
    h = jnp.maximum(cin_ref[...] * agg_ref[0] + b_ref[...], 0.0)
    if residual:
        h = h + res_ref[...]
    h_ref[...] = h
    u_ref[...] = jnp.dot(h * cout_ref[...], w_ref[...],
                         preferred_element_type=jnp.float32)


def _tc_step(agg, cin, cout, b, w_next, res):
    residual = res is not None
    if res is None:
        res = cin
    res_spec = (pl.BlockSpec((R, H), lambda i: (i, 0)) if residual
                else pl.BlockSpec((R, 1), lambda i: (i, 0)))
    return pl.pallas_call(
        functools.partial(_tc_step_body, residual=residual),
        grid=(N // R,),
        in_specs=[
            _AGG_SPEC,
            pl.BlockSpec((R, 1), lambda i: (i, 0)),
            pl.BlockSpec((R, 1), lambda i: (i, 0)),
            pl.BlockSpec((1, H), lambda i: (0, 0)),
            pl.BlockSpec((H, H), lambda i: (0, 0)),
            res_spec,
        ],
        out_specs=[
            pl.BlockSpec((R, H), lambda i: (i, 0)),
            pl.BlockSpec((R, H), lambda i: (i, 0)),
        ],
        out_shape=[
            jax.ShapeDtypeStruct((N, H), jnp.float32),
            jax.ShapeDtypeStruct((N, H), jnp.float32),
        ],
    )(agg, cin, cout, b, w_next, res)


def _tc_last_body(agg_ref, cin_ref, b_ref, res_ref, h_ref, *, residual):
    h = jnp.maximum(cin_ref[...] * agg_ref[0] + b_ref[...], 0.0)
    if residual:
        h = h + res_ref[...]
    h_ref[...] = h


def _tc_last(agg, cin, b, res):
    residual = res is not None
    if res is None:
        res = cin
    res_spec = (pl.BlockSpec((R, H), lambda i: (i, 0)) if residual
                else pl.BlockSpec((R, 1), lambda i: (i, 0)))
    return pl.pallas_call(
        functools.partial(_tc_last_body, residual=residual),
        grid=(N // R,),
        in_specs=[
            _AGG_SPEC,
            pl.BlockSpec((R, 1), lambda i: (i, 0)),
            pl.BlockSpec((1, H), lambda i: (0, 0)),
            res_spec,
        ],
        out_specs=pl.BlockSpec((R, H), lambda i: (i, 0)),
        out_shape=jax.ShapeDtypeStruct((N, H), jnp.float32),
    )(agg, cin, b, res)


def _readout_body(h1_ref, h2_ref, wfc_ref, bfc_ref, out_ref, acc_ref):
    i = pl.program_id(0)

    @pl.when(i == 0)
    def _init():
        acc_ref[...] = jnp.zeros_like(acc_ref)

    acc_ref[...] += jnp.sum(h1_ref[...] + h2_ref[...], axis=0,
                            keepdims=True)

    @pl.when(i == pl.num_programs(0) - 1)
    def _fin():
        readout = acc_ref[...] / (2.0 * N)
        logits = jnp.dot(readout, wfc_ref[...],
                         preferred_element_type=jnp.float32) + bfc_ref[...]
        m = jnp.max(logits, axis=1, keepdims=True)
        z = logits - m
        out_ref[...] = z - jnp.log(jnp.sum(jnp.exp(z), axis=1,
                                           keepdims=True))


def _readout(h1, h2, wfc, bfc):
    out = pl.pallas_call(
        _readout_body,
        grid=(N // R,),
        in_specs=[
            pl.BlockSpec((R, H), lambda i: (i, 0)),
            pl.BlockSpec((R, H), lambda i: (i, 0)),
            pl.BlockSpec((H, N_CLASS), lambda i: (0, 0)),
            pl.BlockSpec((1, N_CLASS), lambda i: (0, 0)),
        ],
        out_specs=pl.BlockSpec((1, N_CLASS), lambda i: (0, 0)),
        out_shape=jax.ShapeDtypeStruct((1, N_CLASS), jnp.float32),
        scratch_shapes=[pltpu.VMEM((1, H), jnp.float32)],
    )(h1, h2, wfc, bfc)
    return out[0]


def kernel(x_angle, x_distance, edge_index, W_in_a, b_in_a, W_hid_a,
           b_hid_a, W_in_d, b_in_d, W_hid_d, b_hid_d, W_fc, b_fc):
    src = edge_index[0]
    dst = edge_index[1]
    deg_out, deg_in = _degrees(src, dst)
    c_out = lax.rsqrt(jnp.clip(deg_out, 1.0))[:, None]
    c_in = lax.rsqrt(jnp.clip(deg_in, 1.0))[:, None]

    idxc, nc = _edge_prep(src, dst)
    spmm_call = _make_spmm()

    def _spmm(u):
        return spmm_call(u, idxc, nc)

    def pad16(x):
        return jnp.pad(x, ((0, 0), (0, 16 - x.shape[1])))

    xa16 = pad16(x_angle)
    xd16 = pad16(x_distance)
    Wina16 = jnp.pad(W_in_a, ((0, 16 - W_in_a.shape[0]), (0, 0)))
    Wind16 = jnp.pad(W_in_d, ((0, 16 - W_in_d.shape[0]), (0, 0)))

    u_a = _tc_in(xa16, c_out, Wina16)
    u_d = _tc_in(xd16, c_out, Wind16)
    h_a = None
    h_d = None
    for p in range(9):
        agg = _spmm(u_a)
        res = h_a if (p >= 1 and (p - 1) not in (3, 7)) else None
        b = b_in_a[None, :] if p == 0 else b_hid_a[p - 1][None, :]
        h_a, u_a = _tc_step(agg, c_in, c_out, b, W_hid_a[p], res)

        agg = _spmm(u_d)
        res = h_d if (p >= 1 and (p - 1) not in (3, 7)) else None
        b = b_in_d[None, :] if p == 0 else b_hid_d[p - 1][None, :]
        if p < 8:
            h_d, u_d = _tc_step(agg, c_in, c_out, b, W_hid_d[p], res)
        else:
            h_d = _tc_last(agg, c_in, b, res)

    for p in range(9, 12):
        agg = _spmm(u_a)
        res = h_a
        b = b_hid_a[p - 1][None, :]
        w_next = W_hid_a[p] if p < 11 else W_hid_d[8]
        h_a, u_a = _tc_step(agg, c_in, c_out, b, w_next, res)

    h1 = h_a
    u_d = u_a
    for p in range(9, 12):
        agg = _spmm(u_d)
        res = h_d
        b = b_hid_d[p - 1][None, :]
        if p < 11:
            h_d, u_d = _tc_step(agg, c_in, c_out, b, W_hid_d[p], res)
        else:
            h_d = _tc_last(agg, c_in, b, res)

    return _readout(h1, h_d, W_fc, b_fc[None, :])
